# baseline (speedup 1.0000x reference)
import functools
import jax
import jax.numpy as jnp
from jax import lax
from jax.experimental import pallas as pl
from jax.experimental.pallas import tpu as pltpu
from jax.experimental.pallas import tpu_sc as plsc

B, L = 4096, 200
NUCL_DIM, SPEC_DIM = 128, 32
OUT_DIM = NUCL_DIM + 2 * SPEC_DIM

NROWS = B * L
NC, NS = 2, 16
NW = NC * NS
ROWS_PER_W = NROWS // NW
CH = 128
NCH = ROWS_PER_W // CH


def _prep_body(x_ref, pbs_ref, rt_ref, nucl_ref, pbst_ref, rtt_ref,
               p2r_ref, nucl8_ref, spec4_ref):
    p = (pbs_ref[...] > 0.5).astype(jnp.int32)
    r = (rt_ref[...] > 0.5).astype(jnp.int32)
    x = x_ref[...]
    p2r_ref[...] = (p * 2 + r) + x * 0
    zeros_row = jnp.zeros((1, NUCL_DIM), jnp.float32)
    nucl8_ref[...] = jnp.concatenate(
        [nucl_ref[...]] + [zeros_row] * 3, axis=0)
    pad = jnp.zeros((1, NUCL_DIM - 2 * SPEC_DIM), jnp.float32)
    rows = []
    for q in range(4):
        rows.append(jnp.concatenate(
            [pbst_ref[q // 2][None, :], rtt_ref[q % 2][None, :], pad],
            axis=1))
    spec4_ref[...] = jnp.concatenate(rows, axis=0)


def _prep(x_seq, pbs_feat, rt_feat, nucl_table, pbs_table, rt_table):
    return pl.pallas_call(
        _prep_body,
        in_specs=[
            pl.BlockSpec((B, L), lambda: (0, 0)),
            pl.BlockSpec((B, 1), lambda: (0, 0)),
            pl.BlockSpec((B, 1), lambda: (0, 0)),
            pl.BlockSpec((5, NUCL_DIM), lambda: (0, 0)),
            pl.BlockSpec((2, SPEC_DIM), lambda: (0, 0)),
            pl.BlockSpec((2, SPEC_DIM), lambda: (0, 0)),
        ],
        out_specs=[
            pl.BlockSpec((B, L), lambda: (0, 0)),
            pl.BlockSpec((8, NUCL_DIM), lambda: (0, 0)),
            pl.BlockSpec((4, NUCL_DIM), lambda: (0, 0)),
        ],
        out_shape=[
            jax.ShapeDtypeStruct((B, L), jnp.int32),
            jax.ShapeDtypeStruct((8, NUCL_DIM), jnp.float32),
            jax.ShapeDtypeStruct((4, NUCL_DIM), jnp.float32),
        ],
    )(x_seq, pbs_feat.reshape(B, 1), rt_feat.reshape(B, 1),
      nucl_table, pbs_table, rt_table)


@functools.partial(
    pl.kernel,
    out_type=jax.ShapeDtypeStruct((NROWS, OUT_DIM), jnp.float32),
    mesh=plsc.VectorSubcoreMesh(core_axis_name="c", subcore_axis_name="s"),
    scratch_types=[
        pltpu.VMEM((CH,), jnp.int32),
        pltpu.VMEM((CH,), jnp.int32),
        pltpu.VMEM((CH, OUT_DIM), jnp.float32),
        pltpu.VMEM((CH, NUCL_DIM), jnp.float32),
        pltpu.SemaphoreType.DMA,
        pltpu.SemaphoreType.DMA,
    ],
)
def _sc_gather(xf_hbm, pf_hbm, nucl8_hbm, spec4_hbm, out_hbm,
               xidx_v, pidx_v, comb, spec_buf, sem1, sem2):
    wid = lax.axis_index("s") * NC + lax.axis_index("c")
    base0 = wid * ROWS_PER_W

    def chunk(i, carry):
        base = base0 + i * CH
        pltpu.sync_copy(xf_hbm.at[pl.ds(base, CH)], xidx_v)
        pltpu.sync_copy(pf_hbm.at[pl.ds(base, CH)], pidx_v)
        cp1 = pltpu.async_copy(nucl8_hbm.at[xidx_v],
                               comb.at[:, pl.ds(0, NUCL_DIM)], sem1)
        cp2 = pltpu.async_copy(spec4_hbm.at[pidx_v], spec_buf, sem2)
        cp1.wait()
        cp2.wait()

        def row_fill(r, c):
            for k in range(2 * SPEC_DIM // 16):
                comb[r, pl.ds(NUCL_DIM + 16 * k, 16)] = \
                    spec_buf[r, pl.ds(16 * k, 16)]
            return c

        lax.fori_loop(0, CH, row_fill, 0)
        pltpu.sync_copy(comb, out_hbm.at[pl.ds(base, CH), :])
        return carry

    lax.fori_loop(0, NCH, chunk, 0)


@jax.jit
def kernel(x_seq, pbs_feat, rt_feat, nucl_table, pbs_table, rt_table):
    p2r, nucl8, spec4 = _prep(x_seq, pbs_feat, rt_feat, nucl_table,
                              pbs_table, rt_table)
    out = _sc_gather(x_seq.reshape(NROWS), p2r.reshape(NROWS), nucl8, spec4)
    return out.reshape(B, L, OUT_DIM)


# speedup vs baseline: 4.4462x; 4.4462x over previous
import functools
import jax
import jax.numpy as jnp
from jax import lax
from jax.experimental import pallas as pl
from jax.experimental.pallas import tpu as pltpu
from jax.experimental.pallas import tpu_sc as plsc

B, L = 4096, 200
NUCL_DIM, SPEC_DIM = 128, 32
OUT_DIM = NUCL_DIM + 2 * SPEC_DIM
PAIR_DIM = 2 * OUT_DIM
NPAIR = B * (L // 2)
TBL = 128

NC, NS = 2, 16
NW = NC * NS
ROWS_PER_W = NPAIR // NW
CH = 128
NCH = ROWS_PER_W // CH


def _prep_body(xe_ref, xo_ref, pbs_ref, rt_ref, nucl_ref, pbst_ref, rtt_ref,
               pidx_ref, tbl_ref):
    p = (pbs_ref[...] > 0.5).astype(jnp.int32)
    r = (rt_ref[...] > 0.5).astype(jnp.int32)
    q = p * 2 + r
    pidx_ref[...] = (xe_ref[...] * 5 + xo_ref[...]) * 4 + q

    c = lax.broadcasted_iota(jnp.int32, (TBL, NUCL_DIM), 0)
    x1 = c // 20
    x2 = (c // 4) % 5
    n1 = jnp.zeros((TBL, NUCL_DIM), jnp.float32)
    n2 = jnp.zeros((TBL, NUCL_DIM), jnp.float32)
    for v in range(5):
        row = nucl_ref[v, :][None, :]
        n1 = jnp.where(x1 == v, row, n1)
        n2 = jnp.where(x2 == v, row, n2)
    cs = lax.broadcasted_iota(jnp.int32, (TBL, SPEC_DIM), 0)
    pb = jnp.where((cs % 4) // 2 == 1, pbst_ref[1, :][None, :],
                   pbst_ref[0, :][None, :])
    rb = jnp.where(cs % 2 == 1, rtt_ref[1, :][None, :],
                   rtt_ref[0, :][None, :])
    tbl_ref[...] = jnp.concatenate([n1, pb, rb, n2, pb, rb], axis=1)


def _prep(xe, xo, pbs_feat, rt_feat, nucl_table, pbs_table, rt_table):
    return pl.pallas_call(
        _prep_body,
        in_specs=[
            pl.BlockSpec((B, L // 2), lambda: (0, 0)),
            pl.BlockSpec((B, L // 2), lambda: (0, 0)),
            pl.BlockSpec((B, 1), lambda: (0, 0)),
            pl.BlockSpec((B, 1), lambda: (0, 0)),
            pl.BlockSpec((5, NUCL_DIM), lambda: (0, 0)),
            pl.BlockSpec((2, SPEC_DIM), lambda: (0, 0)),
            pl.BlockSpec((2, SPEC_DIM), lambda: (0, 0)),
        ],
        out_specs=[
            pl.BlockSpec((B, L // 2), lambda: (0, 0)),
            pl.BlockSpec((TBL, PAIR_DIM), lambda: (0, 0)),
        ],
        out_shape=[
            jax.ShapeDtypeStruct((B, L // 2), jnp.int32),
            jax.ShapeDtypeStruct((TBL, PAIR_DIM), jnp.float32),
        ],
    )(xe, xo, pbs_feat.reshape(B, 1), rt_feat.reshape(B, 1),
      nucl_table, pbs_table, rt_table)


@functools.partial(
    pl.kernel,
    out_type=jax.ShapeDtypeStruct((NPAIR, PAIR_DIM), jnp.float32),
    mesh=plsc.VectorSubcoreMesh(core_axis_name="c", subcore_axis_name="s"),
    scratch_types=[
        pltpu.VMEM((NCH, CH), jnp.int32),
        pltpu.VMEM((CH, PAIR_DIM), jnp.float32),
        pltpu.VMEM((CH, PAIR_DIM), jnp.float32),
        pltpu.SemaphoreType.DMA,
        pltpu.SemaphoreType.DMA,
        pltpu.SemaphoreType.DMA,
        pltpu.SemaphoreType.DMA,
    ],
)
def _sc_gather(pidx_hbm, tbl_hbm, out_hbm,
               idx_all, buf_a, buf_b, gs_a, gs_b, ws_a, ws_b):
    wid = lax.axis_index("s") * NC + lax.axis_index("c")
    base0 = wid * ROWS_PER_W
    pltpu.sync_copy(pidx_hbm.at[wid], idx_all)

    bufs = (buf_a, buf_b)
    gsems = (gs_a, gs_b)
    wsems = (ws_a, ws_b)

    for b in range(2):
        pltpu.async_copy(tbl_hbm.at[idx_all.at[b]], bufs[b], gsems[b]).wait()
        pltpu.async_copy(bufs[b], out_hbm.at[pl.ds(base0 + b * CH, CH)],
                         wsems[b])

    def body(j, carry):
        for b in range(2):
            i = j * 2 + b
            pltpu.make_async_copy(bufs[b], out_hbm.at[pl.ds(0, CH)],
                                  wsems[b]).wait()
            pltpu.async_copy(tbl_hbm.at[idx_all.at[i]], bufs[b],
                             gsems[b]).wait()
            pltpu.async_copy(bufs[b], out_hbm.at[pl.ds(base0 + i * CH, CH)],
                             wsems[b])
        return carry

    lax.fori_loop(1, NCH // 2, body, 0)

    for b in range(2):
        pltpu.make_async_copy(bufs[b], out_hbm.at[pl.ds(0, CH)],
                              wsems[b]).wait()


@jax.jit
def kernel(x_seq, pbs_feat, rt_feat, nucl_table, pbs_table, rt_table):
    xp = x_seq.reshape(B, L // 2, 2)
    pidx, tbl = _prep(xp[:, :, 0], xp[:, :, 1], pbs_feat, rt_feat,
                      nucl_table, pbs_table, rt_table)
    out = _sc_gather(pidx.reshape(NW, NCH, CH), tbl)
    return out.reshape(B, L, OUT_DIM)


# speedup vs baseline: 5.9451x; 1.3371x over previous
import functools
import jax
import jax.numpy as jnp
from jax import lax
from jax.experimental import pallas as pl
from jax.experimental.pallas import tpu as pltpu
from jax.experimental.pallas import tpu_sc as plsc

B, L = 4096, 200
NUCL_DIM, SPEC_DIM = 128, 32
OUT_DIM = NUCL_DIM + 2 * SPEC_DIM
QUAD_DIM = 4 * OUT_DIM
NQUAD = B * (L // 4)
TBL = 2560

NC, NS = 2, 16
NW = NC * NS
ROWS_PER_W = NQUAD // NW
CH = 64
NCH = ROWS_PER_W // CH


def _prep_body(x0_ref, x1_ref, x2_ref, x3_ref, pbs_ref, rt_ref,
               nucl_ref, pbst_ref, rtt_ref, qidx_ref, tbl_ref):
    p = (pbs_ref[...] > 0.5).astype(jnp.int32)
    r = (rt_ref[...] > 0.5).astype(jnp.int32)
    q = p * 2 + r
    qidx_ref[...] = (((x0_ref[...] * 5 + x1_ref[...]) * 5 + x2_ref[...]) * 5
                     + x3_ref[...]) * 4 + q

    c = lax.broadcasted_iota(jnp.int32, (TBL, NUCL_DIM), 0)
    xs = [c // 500, (c // 100) % 5, (c // 20) % 5, (c // 4) % 5]
    ns = []
    for k in range(4):
        n = jnp.zeros((TBL, NUCL_DIM), jnp.float32)
        for v in range(5):
            n = jnp.where(xs[k] == v, nucl_ref[v, :][None, :], n)
        ns.append(n)
    cs = lax.broadcasted_iota(jnp.int32, (TBL, SPEC_DIM), 0)
    pb = jnp.where((cs % 4) // 2 == 1, pbst_ref[1, :][None, :],
                   pbst_ref[0, :][None, :])
    rb = jnp.where(cs % 2 == 1, rtt_ref[1, :][None, :],
                   rtt_ref[0, :][None, :])
    tbl_ref[...] = jnp.concatenate(
        [ns[0], pb, rb, ns[1], pb, rb, ns[2], pb, rb, ns[3], pb, rb], axis=1)


def _prep(x0, x1, x2, x3, pbs_feat, rt_feat, nucl_table, pbs_table, rt_table):
    return pl.pallas_call(
        _prep_body,
        in_specs=[
            pl.BlockSpec((B, L // 4), lambda: (0, 0)),
            pl.BlockSpec((B, L // 4), lambda: (0, 0)),
            pl.BlockSpec((B, L // 4), lambda: (0, 0)),
            pl.BlockSpec((B, L // 4), lambda: (0, 0)),
            pl.BlockSpec((B, 1), lambda: (0, 0)),
            pl.BlockSpec((B, 1), lambda: (0, 0)),
            pl.BlockSpec((5, NUCL_DIM), lambda: (0, 0)),
            pl.BlockSpec((2, SPEC_DIM), lambda: (0, 0)),
            pl.BlockSpec((2, SPEC_DIM), lambda: (0, 0)),
        ],
        out_specs=[
            pl.BlockSpec((B, L // 4), lambda: (0, 0)),
            pl.BlockSpec((TBL, QUAD_DIM), lambda: (0, 0)),
        ],
        out_shape=[
            jax.ShapeDtypeStruct((B, L // 4), jnp.int32),
            jax.ShapeDtypeStruct((TBL, QUAD_DIM), jnp.float32),
        ],
    )(x0, x1, x2, x3, pbs_feat.reshape(B, 1), rt_feat.reshape(B, 1),
      nucl_table, pbs_table, rt_table)


@functools.partial(
    pl.kernel,
    out_type=jax.ShapeDtypeStruct((NQUAD, QUAD_DIM), jnp.float32),
    mesh=plsc.VectorSubcoreMesh(core_axis_name="c", subcore_axis_name="s"),
    scratch_types=[
        pltpu.VMEM((NCH, CH), jnp.int32),
        pltpu.VMEM((CH, QUAD_DIM), jnp.float32),
        pltpu.VMEM((CH, QUAD_DIM), jnp.float32),
        pltpu.SemaphoreType.DMA,
        pltpu.SemaphoreType.DMA,
        pltpu.SemaphoreType.DMA,
        pltpu.SemaphoreType.DMA,
    ],
)
def _sc_gather(qidx_hbm, tbl_hbm, out_hbm,
               idx_all, buf_a, buf_b, gs_a, gs_b, ws_a, ws_b):
    wid = lax.axis_index("s") * NC + lax.axis_index("c")
    base0 = wid * ROWS_PER_W
    pltpu.sync_copy(qidx_hbm.at[wid], idx_all)

    bufs = (buf_a, buf_b)
    gsems = (gs_a, gs_b)
    wsems = (ws_a, ws_b)

    def g_start(i, b):
        pltpu.async_copy(tbl_hbm.at[idx_all.at[i]], bufs[b], gsems[b])

    def g_wait(b):
        pltpu.make_async_copy(tbl_hbm.at[idx_all.at[0]], bufs[b],
                              gsems[b]).wait()

    def w_start(i, b):
        pltpu.async_copy(bufs[b], out_hbm.at[pl.ds(base0 + i * CH, CH)],
                         wsems[b])

    def w_wait(b):
        pltpu.make_async_copy(bufs[b], out_hbm.at[pl.ds(0, CH)],
                              wsems[b]).wait()

    g_start(0, 0)
    g_wait(0)
    g_start(1, 1)
    w_start(0, 0)

    def body(j, carry):
        for b in (1, 0):
            i = 2 * j + (1 if b == 1 else 2)
            g_wait(b)
            w_wait(1 - b)
            g_start(i + 1, 1 - b)
            w_start(i, b)
        return carry

    lax.fori_loop(0, (NCH - 2) // 2, body, 0)

    g_wait(1)
    w_start(NCH - 1, 1)
    w_wait(0)
    w_wait(1)


@jax.jit
def kernel(x_seq, pbs_feat, rt_feat, nucl_table, pbs_table, rt_table):
    xq = x_seq.reshape(B, L // 4, 4)
    qidx, tbl = _prep(xq[:, :, 0], xq[:, :, 1], xq[:, :, 2], xq[:, :, 3],
                      pbs_feat, rt_feat, nucl_table, pbs_table, rt_table)
    out = _sc_gather(qidx.reshape(NW, NCH, CH), tbl)
    return out.reshape(B, L, OUT_DIM)
